# baseline (speedup 1.0000x reference)
import functools

import jax
import jax.numpy as jnp
from jax import lax
from jax.experimental import pallas as pl
from jax.experimental.pallas import tpu as pltpu
from jax.experimental.pallas import tpu_sc as plsc

N = 10000
NP = 10240
D = 128
DH = 64
NC = 2
NS = 16
L = 16
VPR = DH // L

CHUNK = 128
K = 8
ROWS_PER_TILE = NP // NS
SUB = 128
NSUB = ROWS_PER_TILE // SUB
STEPS = 4


def _body(state_hbm, src_hbm, dst_hbm, w_hbm, ap_hbm, out_hbm,
          ax_sh, src_blk, dst_blk, w_blk,
          rows2, sbuf, abuf, avbuf, esem, gsem, ssem, nblk):
    cid = lax.axis_index("c")
    tid = lax.axis_index("s")
    row0 = tid * ROWS_PER_TILE
    hrow0 = cid * NP + row0
    off = cid * NP

    pltpu.sync_copy(ap_hbm, avbuf)
    a = avbuf[...]
    alpha = 1.0 / (1.0 + jnp.exp(-a))

    zv = jnp.zeros((L,), jnp.float32)

    def zrow(r, _):
        for v in range(VPR):
            abuf[r, pl.ds(v * L, L)] = zv
        return 0
    lax.fori_loop(0, SUB, zrow, 0)

    def init_sub(k, _):
        pltpu.sync_copy(state_hbm.at[pl.ds(hrow0 + k * SUB, SUB)], sbuf)
        pltpu.sync_copy(sbuf, out_hbm.at[pl.ds(hrow0 + k * SUB, SUB)])
        pltpu.sync_copy(abuf, ax_sh.at[pl.ds(row0 + k * SUB, SUB)])
        return 0
    lax.fori_loop(0, NSUB, init_sub, 0)
    plsc.subcore_barrier()

    def load_block(bi, eb):
        pltpu.async_copy(src_hbm.at[tid, bi], src_blk.at[eb], esem.at[eb])
        pltpu.async_copy(dst_hbm.at[tid, bi], dst_blk.at[eb], esem.at[eb])
        pltpu.async_copy(w_hbm.at[tid, bi], w_blk.at[eb], esem.at[eb])

    def wait_block(bi, eb):
        pltpu.make_async_copy(src_hbm.at[tid, bi], src_blk.at[eb], esem.at[eb]).wait()
        pltpu.make_async_copy(dst_hbm.at[tid, bi], dst_blk.at[eb], esem.at[eb]).wait()
        pltpu.make_async_copy(w_hbm.at[tid, bi], w_blk.at[eb], esem.at[eb]).wait()

    def step(_, carry):
        load_block(0, 0)

        def block(bi, _):
            eb = lax.bitwise_and(bi, 1)
            wait_block(bi, eb)

            def adj(k, _):
                for g in range(CHUNK // L):
                    sl = pl.ds(g * L, L)
                    src_blk[eb, k, sl] = src_blk[eb, k, sl] + off
                return 0
            lax.fori_loop(0, K, adj, 0)

            load_block(bi + 1, 1 - eb)

            def chunk(cj, _):
                rb = lax.bitwise_and(cj, 1)
                flat = bi * K + cj

                @pl.when(flat >= 1)
                def _():
                    pltpu.make_async_copy(
                        rows2.at[1 - rb], ax_sh.at[dst_blk.at[eb, cj]],
                        ssem.at[1 - rb]).wait()

                @pl.when(cj == 0)
                def _():
                    pltpu.async_copy(
                        out_hbm.at[src_blk.at[eb, 0]], rows2.at[0], gsem.at[0])

                @pl.when(cj < K - 1)
                def _():
                    pltpu.async_copy(
                        out_hbm.at[src_blk.at[eb, cj + 1]],
                        rows2.at[1 - rb], gsem.at[1 - rb])

                pltpu.make_async_copy(
                    out_hbm.at[src_blk.at[eb, cj]], rows2.at[rb],
                    gsem.at[rb]).wait()

                @plsc.parallel_loop(0, CHUNK // L)
                def _mul(g):
                    wvec = w_blk[eb, cj, pl.ds(g * L, L)]
                    for j in range(L):
                        sp = jnp.broadcast_to(wvec[j], (L,))
                        e = g * L + j
                        for v in range(VPR):
                            sl = pl.ds(v * L, L)
                            rows2[rb, e, sl] = rows2[rb, e, sl] * sp

                pltpu.async_copy(rows2.at[rb], ax_sh.at[dst_blk.at[eb, cj]],
                                 ssem.at[rb], add=True)
                return 0
            lax.fori_loop(0, K, chunk, 0)
            return 0
        lax.fori_loop(0, nblk, block, 0)
        wait_block(nblk, lax.bitwise_and(nblk, 1))
        last = lax.bitwise_and(nblk * K - 1, 1)
        pltpu.make_async_copy(
            rows2.at[last], ax_sh.at[dst_blk.at[0, 0]], ssem.at[last]).wait()
        plsc.subcore_barrier()

        def upd_sub(k, _):
            r0 = row0 + k * SUB
            h0 = hrow0 + k * SUB
            pltpu.sync_copy(out_hbm.at[pl.ds(h0, SUB)], sbuf)
            pltpu.sync_copy(ax_sh.at[pl.ds(r0, SUB)], abuf)

            def urow(r, _):
                for v in range(VPR):
                    sl = pl.ds(v * L, L)
                    s = sbuf[r, sl]
                    sbuf[r, sl] = s + alpha * (abuf[r, sl] - s)
                    abuf[r, sl] = zv
                return 0
            lax.fori_loop(0, SUB, urow, 0)

            pltpu.sync_copy(sbuf, out_hbm.at[pl.ds(h0, SUB)])
            pltpu.sync_copy(abuf, ax_sh.at[pl.ds(r0, SUB)])
            return 0
        lax.fori_loop(0, NSUB, upd_sub, 0)
        plsc.subcore_barrier()
        return carry
    lax.fori_loop(0, STEPS, step, 0)


@functools.partial(jax.jit, static_argnames=("nblk",))
def _run(state2, src4, dst4, w4, ap, nblk):
    mesh = plsc.VectorSubcoreMesh(
        core_axis_name="c", subcore_axis_name="s", num_cores=NC, num_subcores=NS
    )
    return pl.kernel(
        functools.partial(_body, nblk=nblk),
        out_type=jax.ShapeDtypeStruct((NC * NP, DH), jnp.float32),
        mesh=mesh,
        compiler_params=pltpu.CompilerParams(use_tc_tiling_on_sc=False),
        scratch_types=[
            pltpu.VMEM_SHARED((NP, DH), jnp.float32),
            pltpu.VMEM((2, K, CHUNK), jnp.int32),
            pltpu.VMEM((2, K, CHUNK), jnp.int32),
            pltpu.VMEM((2, K, CHUNK), jnp.float32),
            pltpu.VMEM((2, CHUNK, DH), jnp.float32),
            pltpu.VMEM((SUB, DH), jnp.float32),
            pltpu.VMEM((SUB, DH), jnp.float32),
            pltpu.VMEM((L,), jnp.float32),
            pltpu.SemaphoreType.DMA((2,)),
            pltpu.SemaphoreType.DMA((2,)),
            pltpu.SemaphoreType.DMA((2,)),
        ],
    )(state2, src4, dst4, w4, ap)


def kernel(x, edge_index, edge_weight, alpha_train):
    E = edge_weight.shape[0]
    blk_edges = K * CHUNK
    nblk = -(-E // (NS * blk_edges))
    ep = nblk * blk_edges * NS
    pad = ep - E

    def layout(arr, dt):
        flat = jnp.concatenate([arr, jnp.zeros((pad,), dt)])
        data = flat.reshape(NS, nblk, K, CHUNK)
        extra = jnp.zeros((NS, 1, K, CHUNK), dt)
        return jnp.concatenate([data, extra], axis=1)

    src4 = layout(edge_index[0], jnp.int32)
    dst4 = layout(edge_index[1], jnp.int32)
    w4 = layout(edge_weight, jnp.float32)

    zpad = jnp.zeros((NP - N, DH), jnp.float32)
    state2 = jnp.concatenate([x[:, :DH], zpad, x[:, DH:], zpad], axis=0)
    ap = jnp.full((L,), alpha_train, jnp.float32)

    out2 = _run(state2, src4, dst4, w4, ap, nblk)
    return jnp.concatenate([out2[:N], out2[NP:NP + N]], axis=1)


# speedup vs baseline: 1.3017x; 1.3017x over previous
import functools

import jax
import jax.numpy as jnp
from jax import lax
from jax.experimental import pallas as pl
from jax.experimental.pallas import tpu as pltpu
from jax.experimental.pallas import tpu_sc as plsc

N = 10000
NP = 10240
D = 128
DH = 64
NC = 2
NS = 16
L = 16
VPR = DH // L

CHUNK = 128
K = 8
ROWS_PER_TILE = NP // NS
SUB = 128
NSUB = ROWS_PER_TILE // SUB
STEPS = 4


def _body(state_hbm, src_hbm, dst_hbm, w_hbm, ap_hbm, out_hbm,
          state_sh, ax_sh, src_blk, dst_blk, w_blk,
          rows2, sbuf, abuf, avbuf, esem, gsem, ssem, nblk):
    cid = lax.axis_index("c")
    tid = lax.axis_index("s")
    row0 = tid * ROWS_PER_TILE
    hrow0 = cid * NP + row0

    pltpu.sync_copy(ap_hbm, avbuf)
    a = avbuf[...]
    alpha = 1.0 / (1.0 + jnp.exp(-a))

    zv = jnp.zeros((L,), jnp.float32)

    def zrow(r, _):
        for v in range(VPR):
            abuf[r, pl.ds(v * L, L)] = zv
        return 0
    lax.fori_loop(0, SUB, zrow, 0)

    def init_sub(k, _):
        pltpu.sync_copy(state_hbm.at[pl.ds(hrow0 + k * SUB, SUB)], sbuf)
        pltpu.sync_copy(sbuf, state_sh.at[pl.ds(row0 + k * SUB, SUB)])
        pltpu.sync_copy(abuf, ax_sh.at[pl.ds(row0 + k * SUB, SUB)])
        return 0
    lax.fori_loop(0, NSUB, init_sub, 0)
    plsc.subcore_barrier()

    def load_block(bi, eb):
        pltpu.async_copy(src_hbm.at[tid, bi], src_blk.at[eb], esem.at[eb])
        pltpu.async_copy(dst_hbm.at[tid, bi], dst_blk.at[eb], esem.at[eb])
        pltpu.async_copy(w_hbm.at[tid, bi], w_blk.at[eb], esem.at[eb])

    def wait_block(bi, eb):
        pltpu.make_async_copy(src_hbm.at[tid, bi], src_blk.at[eb], esem.at[eb]).wait()
        pltpu.make_async_copy(dst_hbm.at[tid, bi], dst_blk.at[eb], esem.at[eb]).wait()
        pltpu.make_async_copy(w_hbm.at[tid, bi], w_blk.at[eb], esem.at[eb]).wait()

    def step(_, carry):
        load_block(0, 0)

        def block(bi, _):
            eb = lax.bitwise_and(bi, 1)

            @pl.when(bi == 0)
            def _():
                wait_block(0, 0)
            load_block(bi + 1, 1 - eb)

            def chunk(cj, _):
                rb = lax.bitwise_and(cj, 1)
                flat = bi * K + cj

                @pl.when(flat >= 1)
                def _():
                    pltpu.make_async_copy(
                        rows2.at[1 - rb], ax_sh.at[dst_blk.at[eb, cj]],
                        ssem.at[1 - rb]).wait()

                @pl.when(flat == 0)
                def _():
                    pltpu.async_copy(
                        state_sh.at[src_blk.at[0, 0]], rows2.at[0], gsem.at[0])

                @pl.when(cj < K - 1)
                def _():
                    pltpu.async_copy(
                        state_sh.at[src_blk.at[eb, cj + 1]],
                        rows2.at[1 - rb], gsem.at[1 - rb])

                @pl.when(cj == K - 1)
                def _():
                    wait_block(bi + 1, 1 - eb)
                    pltpu.async_copy(
                        state_sh.at[src_blk.at[1 - eb, 0]],
                        rows2.at[1 - rb], gsem.at[1 - rb])

                pltpu.make_async_copy(
                    state_sh.at[src_blk.at[eb, cj]], rows2.at[rb],
                    gsem.at[rb]).wait()

                @plsc.parallel_loop(0, CHUNK // L, unroll=2)
                def _mul(g):
                    wvec = w_blk[eb, cj, pl.ds(g * L, L)]
                    for j in range(L):
                        sp = jnp.broadcast_to(wvec[j], (L,))
                        e = g * L + j
                        for v in range(VPR):
                            sl = pl.ds(v * L, L)
                            rows2[rb, e, sl] = rows2[rb, e, sl] * sp

                pltpu.async_copy(rows2.at[rb], ax_sh.at[dst_blk.at[eb, cj]],
                                 ssem.at[rb], add=True)
                return 0
            lax.fori_loop(0, K, chunk, 0)
            return 0
        lax.fori_loop(0, nblk, block, 0)
        last = lax.bitwise_and(nblk * K - 1, 1)
        pltpu.make_async_copy(
            state_sh.at[src_blk.at[0, 0]], rows2.at[1 - last],
            gsem.at[1 - last]).wait()
        pltpu.make_async_copy(
            rows2.at[last], ax_sh.at[dst_blk.at[0, 0]], ssem.at[last]).wait()
        plsc.subcore_barrier()

        def upd_sub(k, _):
            r0 = row0 + k * SUB
            pltpu.sync_copy(state_sh.at[pl.ds(r0, SUB)], sbuf)
            pltpu.sync_copy(ax_sh.at[pl.ds(r0, SUB)], abuf)

            def urow(r, _):
                for v in range(VPR):
                    sl = pl.ds(v * L, L)
                    s = sbuf[r, sl]
                    sbuf[r, sl] = s + alpha * (abuf[r, sl] - s)
                    abuf[r, sl] = zv
                return 0
            lax.fori_loop(0, SUB, urow, 0)

            pltpu.sync_copy(sbuf, state_sh.at[pl.ds(r0, SUB)])
            pltpu.sync_copy(abuf, ax_sh.at[pl.ds(r0, SUB)])
            return 0
        lax.fori_loop(0, NSUB, upd_sub, 0)
        plsc.subcore_barrier()
        return carry
    lax.fori_loop(0, STEPS, step, 0)

    def out_sub(k, _):
        pltpu.sync_copy(state_sh.at[pl.ds(row0 + k * SUB, SUB)], sbuf)
        pltpu.sync_copy(sbuf, out_hbm.at[pl.ds(hrow0 + k * SUB, SUB)])
        return 0
    lax.fori_loop(0, NSUB, out_sub, 0)


@functools.partial(jax.jit, static_argnames=("nblk",))
def _run(state2, src4, dst4, w4, ap, nblk):
    mesh = plsc.VectorSubcoreMesh(
        core_axis_name="c", subcore_axis_name="s", num_cores=NC, num_subcores=NS
    )
    return pl.kernel(
        functools.partial(_body, nblk=nblk),
        out_type=jax.ShapeDtypeStruct((NC * NP, DH), jnp.float32),
        mesh=mesh,
        compiler_params=pltpu.CompilerParams(use_tc_tiling_on_sc=False),
        scratch_types=[
            pltpu.VMEM_SHARED((NP, DH), jnp.float32),
            pltpu.VMEM_SHARED((NP, DH), jnp.float32),
            pltpu.VMEM((2, K, CHUNK), jnp.int32),
            pltpu.VMEM((2, K, CHUNK), jnp.int32),
            pltpu.VMEM((2, K, CHUNK), jnp.float32),
            pltpu.VMEM((2, CHUNK, DH), jnp.float32),
            pltpu.VMEM((SUB, DH), jnp.float32),
            pltpu.VMEM((SUB, DH), jnp.float32),
            pltpu.VMEM((L,), jnp.float32),
            pltpu.SemaphoreType.DMA((2,)),
            pltpu.SemaphoreType.DMA((2,)),
            pltpu.SemaphoreType.DMA((2,)),
        ],
    )(state2, src4, dst4, w4, ap)


def kernel(x, edge_index, edge_weight, alpha_train):
    E = edge_weight.shape[0]
    blk_edges = K * CHUNK
    nblk = -(-E // (NS * blk_edges))
    ep = nblk * blk_edges * NS
    pad = ep - E

    def layout(arr, dt):
        flat = jnp.concatenate([arr, jnp.zeros((pad,), dt)])
        data = flat.reshape(NS, nblk, K, CHUNK)
        extra = jnp.zeros((NS, 1, K, CHUNK), dt)
        return jnp.concatenate([data, extra], axis=1)

    src4 = layout(edge_index[0], jnp.int32)
    dst4 = layout(edge_index[1], jnp.int32)
    w4 = layout(edge_weight, jnp.float32)

    zpad = jnp.zeros((NP - N, DH), jnp.float32)
    state2 = jnp.concatenate([x[:, :DH], zpad, x[:, DH:], zpad], axis=0)
    ap = jnp.full((L,), alpha_train, jnp.float32)

    out2 = _run(state2, src4, dst4, w4, ap, nblk)
    return jnp.concatenate([out2[:N], out2[NP:NP + N]], axis=1)


# speedup vs baseline: 1.4741x; 1.1324x over previous
import functools

import jax
import jax.numpy as jnp
from jax import lax
from jax.experimental import pallas as pl
from jax.experimental.pallas import tpu as pltpu
from jax.experimental.pallas import tpu_sc as plsc

N = 10000
NP = 10240
D = 128
DH = 64
NC = 2
NS = 16
L = 16
VPR = DH // L

CHUNK = 128
K = 8
ROWS_PER_TILE = NP // NS
SUB = 128
NSUB = ROWS_PER_TILE // SUB
STEPS = 4


def _body(state_hbm, src_hbm, dst_hbm, w_hbm, ap_hbm, out_hbm,
          state_sh, ax_sh, src_blk, dst_blk, w_blk,
          rows2, rows_b, sbuf, abuf, pbuf, avbuf, esem, gsem, ssem, nblk):
    cid = lax.axis_index("c")
    tid = lax.axis_index("s")
    row0 = tid * ROWS_PER_TILE
    hrow0 = cid * NP + row0

    pltpu.sync_copy(ap_hbm, avbuf)
    a = avbuf[...]
    alpha = 1.0 / (1.0 + jnp.exp(-a))

    zv = jnp.zeros((L,), jnp.float32)

    def zrow(r, _):
        for v in range(VPR):
            abuf[r, pl.ds(v * L, L)] = zv
        return 0
    lax.fori_loop(0, SUB, zrow, 0)

    def pack_rows(r, _):
        for h in range(VPR // 2):
            a0 = sbuf[r, pl.ds((2 * h) * L, L)]
            a1 = sbuf[r, pl.ds((2 * h + 1) * L, L)]
            pk = plsc.pack(a0, a1, format=plsc.PackFormat.INTERLEAVED)
            pbuf[r, pl.ds(h * L, L)] = plsc.bitcast(pk, jnp.int32)
        return 0

    def init_sub(k, _):
        pltpu.sync_copy(state_hbm.at[pl.ds(hrow0 + k * SUB, SUB)], sbuf)
        pltpu.sync_copy(sbuf, out_hbm.at[pl.ds(hrow0 + k * SUB, SUB)])
        lax.fori_loop(0, SUB, pack_rows, 0)
        pltpu.sync_copy(pbuf, state_sh.at[pl.ds(row0 + k * SUB, SUB)])
        pltpu.sync_copy(abuf, ax_sh.at[pl.ds(row0 + k * SUB, SUB)])
        return 0
    lax.fori_loop(0, NSUB, init_sub, 0)
    plsc.subcore_barrier()

    def load_block(bi, eb):
        pltpu.async_copy(src_hbm.at[tid, bi], src_blk.at[eb], esem.at[eb])
        pltpu.async_copy(dst_hbm.at[tid, bi], dst_blk.at[eb], esem.at[eb])
        pltpu.async_copy(w_hbm.at[tid, bi], w_blk.at[eb], esem.at[eb])

    def wait_block(bi, eb):
        pltpu.make_async_copy(src_hbm.at[tid, bi], src_blk.at[eb], esem.at[eb]).wait()
        pltpu.make_async_copy(dst_hbm.at[tid, bi], dst_blk.at[eb], esem.at[eb]).wait()
        pltpu.make_async_copy(w_hbm.at[tid, bi], w_blk.at[eb], esem.at[eb]).wait()

    def step(_, carry):
        load_block(0, 0)

        def block(bi, _):
            eb = lax.bitwise_and(bi, 1)
            wait_block(bi, eb)
            load_block(bi + 1, 1 - eb)

            def chunk(cj, _):
                rb = lax.bitwise_and(cj, 1)
                flat = bi * K + cj

                @pl.when(flat >= 1)
                def _():
                    pltpu.make_async_copy(
                        rows2.at[1 - rb], ax_sh.at[dst_blk.at[eb, cj]],
                        ssem.at[1 - rb]).wait()

                @pl.when(cj == 0)
                def _():
                    pltpu.async_copy(
                        state_sh.at[src_blk.at[eb, 0]], rows_b.at[0], gsem.at[0])

                @pl.when(cj < K - 1)
                def _():
                    pltpu.async_copy(
                        state_sh.at[src_blk.at[eb, cj + 1]],
                        rows_b.at[1 - rb], gsem.at[1 - rb])

                pltpu.make_async_copy(
                    state_sh.at[src_blk.at[eb, cj]], rows_b.at[rb],
                    gsem.at[rb]).wait()

                @plsc.parallel_loop(0, CHUNK // L)
                def _mul(g):
                    wvec = w_blk[eb, cj, pl.ds(g * L, L)]
                    for j in range(L):
                        sp = jnp.broadcast_to(wvec[j], (L,))
                        e = g * L + j
                        for h in range(VPR // 2):
                            pk = plsc.bitcast(
                                rows_b[rb, e, pl.ds(h * L, L)], jnp.bfloat16)
                            x0, x1 = plsc.unpack(
                                pk, format=plsc.PackFormat.INTERLEAVED)
                            rows2[rb, e, pl.ds((2 * h) * L, L)] = x0 * sp
                            rows2[rb, e, pl.ds((2 * h + 1) * L, L)] = x1 * sp

                pltpu.async_copy(rows2.at[rb], ax_sh.at[dst_blk.at[eb, cj]],
                                 ssem.at[rb], add=True)
                return 0
            lax.fori_loop(0, K, chunk, 0)
            return 0
        lax.fori_loop(0, nblk, block, 0)
        wait_block(nblk, lax.bitwise_and(nblk, 1))
        last = lax.bitwise_and(nblk * K - 1, 1)
        pltpu.make_async_copy(
            rows2.at[last], ax_sh.at[dst_blk.at[0, 0]], ssem.at[last]).wait()
        plsc.subcore_barrier()

        def upd_sub(k, _):
            r0 = row0 + k * SUB
            h0 = hrow0 + k * SUB
            pltpu.sync_copy(out_hbm.at[pl.ds(h0, SUB)], sbuf)
            pltpu.sync_copy(ax_sh.at[pl.ds(r0, SUB)], abuf)

            def urow(r, _):
                for v in range(VPR):
                    sl = pl.ds(v * L, L)
                    s = sbuf[r, sl]
                    sbuf[r, sl] = s + alpha * (abuf[r, sl] - s)
                    abuf[r, sl] = zv
                return 0
            lax.fori_loop(0, SUB, urow, 0)
            lax.fori_loop(0, SUB, pack_rows, 0)

            pltpu.sync_copy(sbuf, out_hbm.at[pl.ds(h0, SUB)])
            pltpu.sync_copy(pbuf, state_sh.at[pl.ds(r0, SUB)])
            pltpu.sync_copy(abuf, ax_sh.at[pl.ds(r0, SUB)])
            return 0
        lax.fori_loop(0, NSUB, upd_sub, 0)
        plsc.subcore_barrier()
        return carry
    lax.fori_loop(0, STEPS, step, 0)


@functools.partial(jax.jit, static_argnames=("nblk",))
def _run(state2, src4, dst4, w4, ap, nblk):
    mesh = plsc.VectorSubcoreMesh(
        core_axis_name="c", subcore_axis_name="s", num_cores=NC, num_subcores=NS
    )
    return pl.kernel(
        functools.partial(_body, nblk=nblk),
        out_type=jax.ShapeDtypeStruct((NC * NP, DH), jnp.float32),
        mesh=mesh,
        compiler_params=pltpu.CompilerParams(
            use_tc_tiling_on_sc=False, needs_layout_passes=False),
        scratch_types=[
            pltpu.VMEM_SHARED((NP, DH // 2), jnp.int32),
            pltpu.VMEM_SHARED((NP, DH), jnp.float32),
            pltpu.VMEM((2, K, CHUNK), jnp.int32),
            pltpu.VMEM((2, K, CHUNK), jnp.int32),
            pltpu.VMEM((2, K, CHUNK), jnp.float32),
            pltpu.VMEM((2, CHUNK, DH), jnp.float32),
            pltpu.VMEM((2, CHUNK, DH // 2), jnp.int32),
            pltpu.VMEM((SUB, DH), jnp.float32),
            pltpu.VMEM((SUB, DH), jnp.float32),
            pltpu.VMEM((SUB, DH // 2), jnp.int32),
            pltpu.VMEM((L,), jnp.float32),
            pltpu.SemaphoreType.DMA((2,)),
            pltpu.SemaphoreType.DMA((2,)),
            pltpu.SemaphoreType.DMA((2,)),
        ],
    )(state2, src4, dst4, w4, ap)


def kernel(x, edge_index, edge_weight, alpha_train):
    E = edge_weight.shape[0]
    blk_edges = K * CHUNK
    nblk = -(-E // (NS * blk_edges))
    ep = nblk * blk_edges * NS
    pad = ep - E

    def layout(arr, dt):
        flat = jnp.concatenate([arr, jnp.zeros((pad,), dt)])
        data = flat.reshape(NS, nblk, K, CHUNK)
        extra = jnp.zeros((NS, 1, K, CHUNK), dt)
        return jnp.concatenate([data, extra], axis=1)

    src4 = layout(edge_index[0], jnp.int32)
    dst4 = layout(edge_index[1], jnp.int32)
    w4 = layout(edge_weight, jnp.float32)

    zpad = jnp.zeros((NP - N, DH), jnp.float32)
    state2 = jnp.concatenate([x[:, :DH], zpad, x[:, DH:], zpad], axis=0)
    ap = jnp.full((L,), alpha_train, jnp.float32)

    out2 = _run(state2, src4, dst4, w4, ap, nblk)
    return jnp.concatenate([out2[:N], out2[NP:NP + N]], axis=1)


# speedup vs baseline: 1.4978x; 1.0161x over previous
import functools

import jax
import jax.numpy as jnp
from jax import lax
from jax.experimental import pallas as pl
from jax.experimental.pallas import tpu as pltpu
from jax.experimental.pallas import tpu_sc as plsc

N = 10000
NP = 10240
D = 128
DH = 64
NC = 2
NS = 16
L = 16
VPR = DH // L

CHUNK = 128
K = 8
ROWS_PER_TILE = NP // NS
SUB = 128
NSUB = ROWS_PER_TILE // SUB
STEPS = 4


def _body(state_hbm, src_hbm, dst_hbm, w_hbm, ap_hbm, out_hbm,
          state_sh, ax_sh, src_blk, dst_blk, w_blk,
          rows2, sbuf, abuf, avbuf, esem, gsem, ssem, nblk):
    cid = lax.axis_index("c")
    tid = lax.axis_index("s")
    row0 = tid * ROWS_PER_TILE
    hrow0 = cid * NP + row0

    pltpu.sync_copy(ap_hbm, avbuf)
    a = avbuf[...]
    alpha = 1.0 / (1.0 + jnp.exp(-a))

    zv = jnp.zeros((L,), jnp.float32)

    def zrow(r, _):
        for v in range(VPR):
            abuf[r, pl.ds(v * L, L)] = zv
        return 0
    lax.fori_loop(0, SUB, zrow, 0)

    def init_sub(k, _):
        pltpu.sync_copy(state_hbm.at[pl.ds(hrow0 + k * SUB, SUB)], sbuf)
        pltpu.sync_copy(sbuf, state_sh.at[pl.ds(row0 + k * SUB, SUB)])
        pltpu.sync_copy(abuf, ax_sh.at[pl.ds(row0 + k * SUB, SUB)])
        return 0
    lax.fori_loop(0, NSUB, init_sub, 0)
    plsc.subcore_barrier()

    def load_block(bi, eb):
        pltpu.async_copy(src_hbm.at[tid, bi], src_blk.at[eb], esem.at[eb])
        pltpu.async_copy(dst_hbm.at[tid, bi], dst_blk.at[eb], esem.at[eb])
        pltpu.async_copy(w_hbm.at[tid, bi], w_blk.at[eb], esem.at[eb])

    def wait_block(bi, eb):
        pltpu.make_async_copy(src_hbm.at[tid, bi], src_blk.at[eb], esem.at[eb]).wait()
        pltpu.make_async_copy(dst_hbm.at[tid, bi], dst_blk.at[eb], esem.at[eb]).wait()
        pltpu.make_async_copy(w_hbm.at[tid, bi], w_blk.at[eb], esem.at[eb]).wait()

    def step(_, carry):
        load_block(0, 0)

        def block(bi, _):
            eb = lax.bitwise_and(bi, 1)
            wait_block(bi, eb)
            load_block(bi + 1, 1 - eb)

            def chunk(cj, _):
                rb = lax.bitwise_and(cj, 1)
                flat = bi * K + cj

                @pl.when(flat >= 1)
                def _():
                    pltpu.make_async_copy(
                        rows2.at[1 - rb], ax_sh.at[dst_blk.at[eb, cj]],
                        ssem.at[1 - rb]).wait()

                @pl.when(cj == 0)
                def _():
                    pltpu.async_copy(
                        state_sh.at[src_blk.at[eb, 0]], rows2.at[0], gsem.at[0])

                @pl.when(cj < K - 1)
                def _():
                    pltpu.async_copy(
                        state_sh.at[src_blk.at[eb, cj + 1]],
                        rows2.at[1 - rb], gsem.at[1 - rb])

                pltpu.make_async_copy(
                    state_sh.at[src_blk.at[eb, cj]], rows2.at[rb],
                    gsem.at[rb]).wait()

                @plsc.parallel_loop(0, CHUNK // L)
                def _mul(g):
                    wvec = w_blk[eb, cj, pl.ds(g * L, L)]
                    for j in range(L):
                        sp = jnp.broadcast_to(wvec[j], (L,))
                        e = g * L + j
                        for v in range(VPR):
                            sl = pl.ds(v * L, L)
                            rows2[rb, e, sl] = rows2[rb, e, sl] * sp

                pltpu.async_copy(rows2.at[rb], ax_sh.at[dst_blk.at[eb, cj]],
                                 ssem.at[rb], add=True)
                return 0
            lax.fori_loop(0, K, chunk, 0)
            return 0
        lax.fori_loop(0, nblk, block, 0)
        wait_block(nblk, lax.bitwise_and(nblk, 1))
        last = lax.bitwise_and(nblk * K - 1, 1)
        pltpu.make_async_copy(
            rows2.at[last], ax_sh.at[dst_blk.at[0, 0]], ssem.at[last]).wait()
        plsc.subcore_barrier()

        def upd_sub(k, _):
            r0 = row0 + k * SUB
            pltpu.sync_copy(state_sh.at[pl.ds(r0, SUB)], sbuf)
            pltpu.sync_copy(ax_sh.at[pl.ds(r0, SUB)], abuf)

            def urow(r, _):
                for v in range(VPR):
                    sl = pl.ds(v * L, L)
                    s = sbuf[r, sl]
                    sbuf[r, sl] = s + alpha * (abuf[r, sl] - s)
                    abuf[r, sl] = zv
                return 0
            lax.fori_loop(0, SUB, urow, 0)

            pltpu.sync_copy(sbuf, state_sh.at[pl.ds(r0, SUB)])
            pltpu.sync_copy(abuf, ax_sh.at[pl.ds(r0, SUB)])
            return 0
        lax.fori_loop(0, NSUB, upd_sub, 0)
        plsc.subcore_barrier()
        return carry
    lax.fori_loop(0, STEPS, step, 0)

    def out_sub(k, _):
        pltpu.sync_copy(state_sh.at[pl.ds(row0 + k * SUB, SUB)], sbuf)
        pltpu.sync_copy(sbuf, out_hbm.at[pl.ds(hrow0 + k * SUB, SUB)])
        return 0
    lax.fori_loop(0, NSUB, out_sub, 0)


@functools.partial(jax.jit, static_argnames=("nblk",))
def _run(state2, src4, dst4, w4, ap, nblk):
    mesh = plsc.VectorSubcoreMesh(
        core_axis_name="c", subcore_axis_name="s", num_cores=NC, num_subcores=NS
    )
    return pl.kernel(
        functools.partial(_body, nblk=nblk),
        out_type=jax.ShapeDtypeStruct((NC * NP, DH), jnp.float32),
        mesh=mesh,
        compiler_params=pltpu.CompilerParams(use_tc_tiling_on_sc=False),
        scratch_types=[
            pltpu.VMEM_SHARED((NP, DH), jnp.float32),
            pltpu.VMEM_SHARED((NP, DH), jnp.float32),
            pltpu.VMEM((2, K, CHUNK), jnp.int32),
            pltpu.VMEM((2, K, CHUNK), jnp.int32),
            pltpu.VMEM((2, K, CHUNK), jnp.float32),
            pltpu.VMEM((2, CHUNK, DH), jnp.float32),
            pltpu.VMEM((SUB, DH), jnp.float32),
            pltpu.VMEM((SUB, DH), jnp.float32),
            pltpu.VMEM((L,), jnp.float32),
            pltpu.SemaphoreType.DMA((2,)),
            pltpu.SemaphoreType.DMA((2,)),
            pltpu.SemaphoreType.DMA((2,)),
        ],
    )(state2, src4, dst4, w4, ap)


def kernel(x, edge_index, edge_weight, alpha_train):
    E = edge_weight.shape[0]
    blk_edges = K * CHUNK
    nblk = -(-E // (NS * blk_edges))
    ep = nblk * blk_edges * NS
    pad = ep - E

    def layout(arr, dt):
        flat = jnp.concatenate([arr, jnp.zeros((pad,), dt)])
        data = flat.reshape(NS, nblk, K, CHUNK)
        extra = jnp.zeros((NS, 1, K, CHUNK), dt)
        return jnp.concatenate([data, extra], axis=1)

    src4 = layout(edge_index[0], jnp.int32)
    dst4 = layout(edge_index[1], jnp.int32)
    w4 = layout(edge_weight, jnp.float32)

    zpad = jnp.zeros((NP - N, DH), jnp.float32)
    state2 = jnp.concatenate([x[:, :DH], zpad, x[:, DH:], zpad], axis=0)
    ap = jnp.full((L,), alpha_train, jnp.float32)

    out2 = _run(state2, src4, dst4, w4, ap, nblk)
    return jnp.concatenate([out2[:N], out2[NP:NP + N]], axis=1)
